# Initial kernel scaffold; baseline (speedup 1.0000x reference)
import jax
import jax.numpy as jnp
from jax.experimental import pallas as pl


def kernel(feature, structure, edge_index, W1, b1, W2, b2, W3, b3, emb, embW, alpha_s, alpha_a):
    raise NotImplementedError("write your pallas kernel here")


# speedup vs baseline: 3.3765x; 3.3765x over previous
import functools

import jax
import jax.numpy as jnp
from jax import lax
from jax.experimental import pallas as pl
from jax.experimental.pallas import tpu as pltpu
from jax.experimental.pallas import tpu_sc as plsc

_N = 50000
_E = 800000
_SUB = 64
_NSUB = 8
_E2 = 819200
_NR = _E2 // _SUB
_DEGPAD = 51200
_Q = 12544
_DUMP = 128
_AR = _Q + _DUMP
_DSTPAD = 50200

_MESH = plsc.VectorSubcoreMesh(core_axis_name="c", subcore_axis_name="s")


@functools.partial(
    pl.kernel,
    out_type=jax.ShapeDtypeStruct((2 * _DEGPAD,), jnp.float32),
    mesh=_MESH,
    scratch_types=[
        pltpu.VMEM((_NSUB, _SUB), jnp.int32),
        pltpu.VMEM((_SUB,), jnp.float32),
        pltpu.VMEM((3200,), jnp.float32),
        pltpu.VMEM_SHARED((_DEGPAD,), jnp.float32),
        pltpu.SemaphoreType.DMA,
    ],
)
def _deg_kernel(dst2_hbm, deg_hbm, didx_v, ones_v, zbuf_v, deg_sh, ssem):
    cid = lax.axis_index("c")
    sid = lax.axis_index("s")
    z = jnp.zeros((16,), jnp.float32)

    def fz(i, _):
        zbuf_v[pl.ds(i * 16, 16)] = z
        return 0

    lax.fori_loop(0, 200, fz, 0)
    o = jnp.full((16,), 1.0, jnp.float32)
    for i in range(_SUB // 16):
        ones_v[pl.ds(i * 16, 16)] = o
    pltpu.sync_copy(zbuf_v, deg_sh.at[pl.ds(sid * 3200, 3200)])
    plsc.subcore_barrier()

    def block(b, _):
        row0 = cid * (_NR // 2) + sid * (_NR // 32) + b * _NSUB
        pltpu.sync_copy(dst2_hbm.at[pl.ds(row0, _NSUB)], didx_v)
        descs = [
            pltpu.async_copy(ones_v.at[pl.ds(0, _SUB)],
                             deg_sh.at[didx_v.at[j]], ssem, add=True)
            for j in range(_NSUB)
        ]
        for d in descs:
            d.wait()
        return 0

    lax.fori_loop(0, _NR // 32 // _NSUB, block, 0)
    plsc.subcore_barrier()
    pltpu.sync_copy(deg_sh.at[pl.ds(sid * 3200, 3200)],
                    deg_hbm.at[pl.ds(cid * _DEGPAD + sid * 3200, 3200)])


@functools.partial(
    pl.kernel,
    out_type=jax.ShapeDtypeStruct((4 * _Q, 128), jnp.float32),
    mesh=_MESH,
    scratch_types=[
        pltpu.VMEM((_NSUB, _SUB), jnp.int32),
        pltpu.VMEM((_NSUB, _SUB), jnp.int32),
        pltpu.VMEM((_NSUB, _SUB), jnp.int32),
        pltpu.VMEM((2, _SUB, 128), jnp.float32),
        pltpu.VMEM_SHARED((_AR, 128), jnp.float32),
        pltpu.SemaphoreType.DMA,
        pltpu.SemaphoreType.DMA,
    ],
)
def _mp_kernel(src2_hbm, dst2_hbm, g_hbm, zeros_hbm, out_hbm,
               sidx_v, didx_v, didx2_v, rows_v, acc_sh, gsem, ssem):
    cid = lax.axis_index("c")
    sid = lax.axis_index("s")

    for rnd in range(2):
        base = (cid * 2 + rnd) * _Q
        pltpu.sync_copy(zeros_hbm, acc_sh.at[pl.ds(sid * 792, 792)])
        plsc.subcore_barrier()

        def block(b, _):
            row0 = sid * (_NR // 16) + b * _NSUB
            pltpu.sync_copy(src2_hbm.at[pl.ds(row0, _NSUB)], sidx_v)
            pltpu.sync_copy(dst2_hbm.at[pl.ds(row0, _NSUB)], didx_v)
            for j in range(_NSUB):
                for c in range(_SUB // 16):
                    v = didx_v[j, pl.ds(c * 16, 16)]
                    local = v - base
                    oob = (local < 0) | (local >= _Q)
                    dump = _Q + (v & (_DUMP - 1))
                    didx2_v[j, pl.ds(c * 16, 16)] = jnp.where(oob, dump, local)
            g_d = [None, None]
            s_d = [None, None]
            for j in range(_NSUB):
                p = j % 2
                if s_d[p] is not None:
                    s_d[p].wait()
                g_d[p] = pltpu.async_copy(
                    g_hbm.at[sidx_v.at[j]], rows_v.at[p], gsem)
                g_d[p].wait()
                s_d[p] = pltpu.async_copy(
                    rows_v.at[p], acc_sh.at[didx2_v.at[j]], ssem, add=True)
            s_d[0].wait()
            s_d[1].wait()
            return 0

        lax.fori_loop(0, _NR // 16 // _NSUB, block, 0)
        plsc.subcore_barrier()
        pltpu.sync_copy(acc_sh.at[pl.ds(sid * 784, 784)],
                        out_hbm.at[pl.ds(base + sid * 784, 784)])
        plsc.subcore_barrier()


_RB = 1000


def _tc1_body(feat_ref, struc_ref, dega_ref, degb_ref, w1_ref, w2_ref,
              emb_ref, embw_ref, g_ref, dis_ref):
    deg = dega_ref[...] + degb_ref[...] + 1.0
    dis = lax.rsqrt(deg)
    gate_row = lax.dot_general(embw_ref[...], emb_ref[...],
                               (((0,), (1,)), ((), ())))
    mask = jnp.where(jax.nn.sigmoid(gate_row) < 0.5, 0.0, 1.0)
    h1 = jnp.dot(feat_ref[...], w1_ref[...], preferred_element_type=jnp.float32)
    h2 = jnp.dot(struc_ref[...] * mask, w2_ref[...],
                 preferred_element_type=jnp.float32)
    g_ref[...] = dis * jnp.concatenate([h1, h2], axis=1)
    dis_ref[...] = dis


def _tc2_body(acc_ref, g_ref, dis_ref, b1_ref, b2_ref,
              feat_ref, struc_ref, g2_ref):
    dis = dis_ref[...]
    s = dis * (acc_ref[...] + g_ref[...])
    feat = s[:, :64] + b1_ref[...]
    struc = s[:, 64:] + b2_ref[...]
    feat_ref[...] = feat
    struc_ref[...] = struc
    g2_ref[...] = dis * jnp.concatenate([feat, struc], axis=1)


def _tc3_body(acc_ref, g2_ref, dis_ref, w3_ref, b3_ref, y_ref):
    s = dis_ref[...] * (acc_ref[...] + g2_ref[...])
    y_ref[...] = jnp.dot(s, w3_ref[...],
                         preferred_element_type=jnp.float32) + b3_ref[...]


def _row_spec(cols):
    return pl.BlockSpec((_RB, cols), lambda i: (i, 0))


def _full_spec(r, c):
    return pl.BlockSpec((r, c), lambda i: (0, 0))


def kernel(feature, structure, edge_index, W1, b1, W2, b2, W3, b3, emb, embW,
           alpha_s, alpha_a):
    pad = _E2 - _E
    src2 = jnp.concatenate(
        [edge_index[0], jnp.zeros((pad,), jnp.int32)]).reshape(_NR, _SUB)
    dst2 = jnp.concatenate(
        [edge_index[1], jnp.full((pad,), _DSTPAD, jnp.int32)]).reshape(_NR, _SUB)
    zeros_init = jnp.zeros((792, 128), jnp.float32)
    structure_p = jnp.pad(structure, ((0, 0), (0, 2)))
    W2p = jnp.pad(W2, ((0, 2), (0, 0)))
    embp = jnp.pad(emb, ((0, 2), (0, 0)))
    total = jnp.abs(alpha_s) + jnp.abs(alpha_a)
    W3s = jnp.concatenate([W3[:64] * (2.0 * alpha_a / total),
                           W3[64:] * (2.0 * alpha_s / total)], axis=0)

    deg_r = _deg_kernel(dst2)
    dega = deg_r[:_N].reshape(_N, 1)
    degb = deg_r[_DEGPAD:_DEGPAD + _N].reshape(_N, 1)

    g, dis = pl.pallas_call(
        _tc1_body,
        grid=(_N // _RB,),
        in_specs=[
            _row_spec(128), _row_spec(8), _row_spec(1), _row_spec(1),
            _full_spec(128, 64), _full_spec(8, 64), _full_spec(8, 64),
            _full_spec(64, 1),
        ],
        out_specs=[_row_spec(128), _row_spec(1)],
        out_shape=[jax.ShapeDtypeStruct((_N, 128), jnp.float32),
                   jax.ShapeDtypeStruct((_N, 1), jnp.float32)],
    )(feature, structure_p, dega, degb, W1, W2p, embp, embW)

    acc1 = _mp_kernel(src2, dst2, g, zeros_init)

    feat_out, struc_out, g2 = pl.pallas_call(
        _tc2_body,
        grid=(_N // _RB,),
        in_specs=[_row_spec(128), _row_spec(128), _row_spec(1),
                  _full_spec(1, 64), _full_spec(1, 64)],
        out_specs=[_row_spec(64), _row_spec(64), _row_spec(128)],
        out_shape=[
            jax.ShapeDtypeStruct((_N, 64), jnp.float32),
            jax.ShapeDtypeStruct((_N, 64), jnp.float32),
            jax.ShapeDtypeStruct((_N, 128), jnp.float32),
        ],
    )(acc1, g, dis, b1.reshape(1, 64), b2.reshape(1, 64))

    acc2 = _mp_kernel(src2, dst2, g2, zeros_init)

    y = pl.pallas_call(
        _tc3_body,
        grid=(_N // _RB,),
        in_specs=[_row_spec(128), _row_spec(128), _row_spec(1),
                  _full_spec(128, 16), _full_spec(1, 16)],
        out_specs=_row_spec(16),
        out_shape=jax.ShapeDtypeStruct((_N, 16), jnp.float32),
    )(acc2, g2, dis, W3s, b3.reshape(1, 16))

    return (feat_out, struc_out, y)


# speedup vs baseline: 3.6419x; 1.0786x over previous
import functools

import jax
import jax.numpy as jnp
from jax import lax
from jax.experimental import pallas as pl
from jax.experimental.pallas import tpu as pltpu
from jax.experimental.pallas import tpu_sc as plsc

_N = 50000
_E = 800000
_SUB = 64
_NSUB = 8
_E2 = 819200
_NR = _E2 // _SUB
_DEGPAD = 51200
_Q = 12544
_DUMP = 128
_AR = _Q + _DUMP
_DSTPAD = 50200

_MESH = plsc.VectorSubcoreMesh(core_axis_name="c", subcore_axis_name="s")


@functools.partial(
    pl.kernel,
    out_type=jax.ShapeDtypeStruct((2 * _DEGPAD,), jnp.float32),
    mesh=_MESH,
    scratch_types=[
        pltpu.VMEM((_NSUB, _SUB), jnp.int32),
        pltpu.VMEM((_SUB,), jnp.float32),
        pltpu.VMEM((3200,), jnp.float32),
        pltpu.VMEM_SHARED((_DEGPAD,), jnp.float32),
        pltpu.SemaphoreType.DMA,
    ],
)
def _deg_kernel(dst2_hbm, deg_hbm, didx_v, ones_v, zbuf_v, deg_sh, ssem):
    cid = lax.axis_index("c")
    sid = lax.axis_index("s")
    z = jnp.zeros((16,), jnp.float32)

    def fz(i, _):
        zbuf_v[pl.ds(i * 16, 16)] = z
        return 0

    lax.fori_loop(0, 200, fz, 0)
    o = jnp.full((16,), 1.0, jnp.float32)
    for i in range(_SUB // 16):
        ones_v[pl.ds(i * 16, 16)] = o
    pltpu.sync_copy(zbuf_v, deg_sh.at[pl.ds(sid * 3200, 3200)])
    plsc.subcore_barrier()

    def block(b, _):
        row0 = cid * (_NR // 2) + sid * (_NR // 32) + b * _NSUB
        pltpu.sync_copy(dst2_hbm.at[pl.ds(row0, _NSUB)], didx_v)
        descs = [
            pltpu.async_copy(ones_v.at[pl.ds(0, _SUB)],
                             deg_sh.at[didx_v.at[j]], ssem, add=True)
            for j in range(_NSUB)
        ]
        for d in descs:
            d.wait()
        return 0

    lax.fori_loop(0, _NR // 32 // _NSUB, block, 0)
    plsc.subcore_barrier()
    pltpu.sync_copy(deg_sh.at[pl.ds(sid * 3200, 3200)],
                    deg_hbm.at[pl.ds(cid * _DEGPAD + sid * 3200, 3200)])


@functools.partial(
    pl.kernel,
    out_type=jax.ShapeDtypeStruct((4 * _Q, 128), jnp.float32),
    mesh=_MESH,
    scratch_types=[
        pltpu.VMEM((_NSUB, _SUB), jnp.int32),
        pltpu.VMEM((_NSUB, _SUB), jnp.int32),
        pltpu.VMEM((_NSUB, _SUB), jnp.int32),
        pltpu.VMEM((3, _SUB, 128), jnp.float32),
        pltpu.VMEM_SHARED((_AR, 128), jnp.float32),
        pltpu.SemaphoreType.DMA,
        pltpu.SemaphoreType.DMA,
        pltpu.SemaphoreType.DMA,
        pltpu.SemaphoreType.DMA,
        pltpu.SemaphoreType.DMA,
        pltpu.SemaphoreType.DMA,
    ],
)
def _mp_kernel(src2_hbm, dst2_hbm, g_hbm, zeros_hbm, out_hbm,
               sidx_v, didx_v, didx2_v, rows_v, acc_sh,
               gsem0, gsem1, gsem2, ssem0, ssem1, ssem2):
    cid = lax.axis_index("c")
    sid = lax.axis_index("s")
    gsems = [gsem0, gsem1, gsem2]
    ssems = [ssem0, ssem1, ssem2]

    for rnd in range(2):
        base = (cid * 2 + rnd) * _Q
        pltpu.sync_copy(zeros_hbm, acc_sh.at[pl.ds(sid * 792, 792)])
        plsc.subcore_barrier()

        def block(b, _):
            row0 = sid * (_NR // 16) + b * _NSUB
            pltpu.sync_copy(src2_hbm.at[pl.ds(row0, _NSUB)], sidx_v)
            pltpu.sync_copy(dst2_hbm.at[pl.ds(row0, _NSUB)], didx_v)
            for j in range(_NSUB):
                for c in range(_SUB // 16):
                    v = didx_v[j, pl.ds(c * 16, 16)]
                    local = v - base
                    oob = (local < 0) | (local >= _Q)
                    dump = _Q + (v & (_DUMP - 1))
                    didx2_v[j, pl.ds(c * 16, 16)] = jnp.where(oob, dump, local)
            g_d = [None] * 3
            s_d = [None] * 3
            for j in range(_NSUB + 1):
                if j < _NSUB:
                    p = j % 3
                    if s_d[p] is not None:
                        s_d[p].wait()
                    g_d[p] = pltpu.async_copy(
                        g_hbm.at[sidx_v.at[j]], rows_v.at[p], gsems[p])
                if j >= 1:
                    q = (j - 1) % 3
                    g_d[q].wait()
                    s_d[q] = pltpu.async_copy(
                        rows_v.at[q], acc_sh.at[didx2_v.at[j - 1]],
                        ssems[q], add=True)
            for d in s_d:
                if d is not None:
                    d.wait()
            return 0

        lax.fori_loop(0, _NR // 16 // _NSUB, block, 0)
        plsc.subcore_barrier()
        pltpu.sync_copy(acc_sh.at[pl.ds(sid * 784, 784)],
                        out_hbm.at[pl.ds(base + sid * 784, 784)])
        plsc.subcore_barrier()


_RB = 1000


def _tc1_body(feat_ref, struc_ref, dega_ref, degb_ref, w1_ref, w2_ref,
              emb_ref, embw_ref, g_ref, dis_ref):
    deg = dega_ref[...] + degb_ref[...] + 1.0
    dis = lax.rsqrt(deg)
    gate_row = lax.dot_general(embw_ref[...], emb_ref[...],
                               (((0,), (1,)), ((), ())))
    mask = jnp.where(jax.nn.sigmoid(gate_row) < 0.5, 0.0, 1.0)
    h1 = jnp.dot(feat_ref[...], w1_ref[...], preferred_element_type=jnp.float32)
    h2 = jnp.dot(struc_ref[...] * mask, w2_ref[...],
                 preferred_element_type=jnp.float32)
    g_ref[...] = dis * jnp.concatenate([h1, h2], axis=1)
    dis_ref[...] = dis


def _tc2_body(acc_ref, g_ref, dis_ref, b1_ref, b2_ref,
              feat_ref, struc_ref, g2_ref):
    dis = dis_ref[...]
    s = dis * (acc_ref[...] + g_ref[...])
    feat = s[:, :64] + b1_ref[...]
    struc = s[:, 64:] + b2_ref[...]
    feat_ref[...] = feat
    struc_ref[...] = struc
    g2_ref[...] = dis * jnp.concatenate([feat, struc], axis=1)


def _tc3_body(acc_ref, g2_ref, dis_ref, w3_ref, b3_ref, y_ref):
    s = dis_ref[...] * (acc_ref[...] + g2_ref[...])
    y_ref[...] = jnp.dot(s, w3_ref[...],
                         preferred_element_type=jnp.float32) + b3_ref[...]


def _row_spec(cols):
    return pl.BlockSpec((_RB, cols), lambda i: (i, 0))


def _full_spec(r, c):
    return pl.BlockSpec((r, c), lambda i: (0, 0))


def kernel(feature, structure, edge_index, W1, b1, W2, b2, W3, b3, emb, embW,
           alpha_s, alpha_a):
    pad = _E2 - _E
    src2 = jnp.concatenate(
        [edge_index[0], jnp.zeros((pad,), jnp.int32)]).reshape(_NR, _SUB)
    dst2 = jnp.concatenate(
        [edge_index[1], jnp.full((pad,), _DSTPAD, jnp.int32)]).reshape(_NR, _SUB)
    zeros_init = jnp.zeros((792, 128), jnp.float32)
    structure_p = jnp.pad(structure, ((0, 0), (0, 2)))
    W2p = jnp.pad(W2, ((0, 2), (0, 0)))
    embp = jnp.pad(emb, ((0, 2), (0, 0)))
    total = jnp.abs(alpha_s) + jnp.abs(alpha_a)
    W3s = jnp.concatenate([W3[:64] * (2.0 * alpha_a / total),
                           W3[64:] * (2.0 * alpha_s / total)], axis=0)

    deg_r = _deg_kernel(dst2)
    dega = deg_r[:_N].reshape(_N, 1)
    degb = deg_r[_DEGPAD:_DEGPAD + _N].reshape(_N, 1)

    g, dis = pl.pallas_call(
        _tc1_body,
        grid=(_N // _RB,),
        in_specs=[
            _row_spec(128), _row_spec(8), _row_spec(1), _row_spec(1),
            _full_spec(128, 64), _full_spec(8, 64), _full_spec(8, 64),
            _full_spec(64, 1),
        ],
        out_specs=[_row_spec(128), _row_spec(1)],
        out_shape=[jax.ShapeDtypeStruct((_N, 128), jnp.float32),
                   jax.ShapeDtypeStruct((_N, 1), jnp.float32)],
    )(feature, structure_p, dega, degb, W1, W2p, embp, embW)

    acc1 = _mp_kernel(src2, dst2, g, zeros_init)

    feat_out, struc_out, g2 = pl.pallas_call(
        _tc2_body,
        grid=(_N // _RB,),
        in_specs=[_row_spec(128), _row_spec(128), _row_spec(1),
                  _full_spec(1, 64), _full_spec(1, 64)],
        out_specs=[_row_spec(64), _row_spec(64), _row_spec(128)],
        out_shape=[
            jax.ShapeDtypeStruct((_N, 64), jnp.float32),
            jax.ShapeDtypeStruct((_N, 64), jnp.float32),
            jax.ShapeDtypeStruct((_N, 128), jnp.float32),
        ],
    )(acc1, g, dis, b1.reshape(1, 64), b2.reshape(1, 64))

    acc2 = _mp_kernel(src2, dst2, g2, zeros_init)

    y = pl.pallas_call(
        _tc3_body,
        grid=(_N // _RB,),
        in_specs=[_row_spec(128), _row_spec(128), _row_spec(1),
                  _full_spec(128, 16), _full_spec(1, 16)],
        out_specs=_row_spec(16),
        out_shape=jax.ShapeDtypeStruct((_N, 16), jnp.float32),
    )(acc2, g2, dis, W3s, b3.reshape(1, 16))

    return (feat_out, struc_out, y)


# speedup vs baseline: 3.7606x; 1.0326x over previous
import functools

import jax
import jax.numpy as jnp
from jax import lax
from jax.experimental import pallas as pl
from jax.experimental.pallas import tpu as pltpu
from jax.experimental.pallas import tpu_sc as plsc

_N = 50000
_E = 800000
_SUB = 64
_NSUB = 16
_E2 = 819200
_NR = _E2 // _SUB
_DEGPAD = 51200
_Q = 12544
_DUMP = 128
_AR = _Q + _DUMP
_DSTPAD = 50200

_MESH = plsc.VectorSubcoreMesh(core_axis_name="c", subcore_axis_name="s")


@functools.partial(
    pl.kernel,
    out_type=jax.ShapeDtypeStruct((2 * _DEGPAD,), jnp.float32),
    mesh=_MESH,
    scratch_types=[
        pltpu.VMEM((_NSUB, _SUB), jnp.int32),
        pltpu.VMEM((_SUB,), jnp.float32),
        pltpu.VMEM((3200,), jnp.float32),
        pltpu.VMEM_SHARED((_DEGPAD,), jnp.float32),
        pltpu.SemaphoreType.DMA,
    ],
)
def _deg_kernel(dst2_hbm, deg_hbm, didx_v, ones_v, zbuf_v, deg_sh, ssem):
    cid = lax.axis_index("c")
    sid = lax.axis_index("s")
    z = jnp.zeros((16,), jnp.float32)

    def fz(i, _):
        zbuf_v[pl.ds(i * 16, 16)] = z
        return 0

    lax.fori_loop(0, 200, fz, 0)
    o = jnp.full((16,), 1.0, jnp.float32)
    for i in range(_SUB // 16):
        ones_v[pl.ds(i * 16, 16)] = o
    pltpu.sync_copy(zbuf_v, deg_sh.at[pl.ds(sid * 3200, 3200)])
    plsc.subcore_barrier()

    def block(b, _):
        row0 = cid * (_NR // 2) + sid * (_NR // 32) + b * _NSUB
        pltpu.sync_copy(dst2_hbm.at[pl.ds(row0, _NSUB)], didx_v)
        descs = [
            pltpu.async_copy(ones_v.at[pl.ds(0, _SUB)],
                             deg_sh.at[didx_v.at[j]], ssem, add=True)
            for j in range(_NSUB)
        ]
        for d in descs:
            d.wait()
        return 0

    lax.fori_loop(0, _NR // 32 // _NSUB, block, 0)
    plsc.subcore_barrier()
    pltpu.sync_copy(deg_sh.at[pl.ds(sid * 3200, 3200)],
                    deg_hbm.at[pl.ds(cid * _DEGPAD + sid * 3200, 3200)])


@functools.partial(
    pl.kernel,
    out_type=jax.ShapeDtypeStruct((4 * _Q, 128), jnp.float32),
    mesh=_MESH,
    scratch_types=[
        pltpu.VMEM((_NSUB, _SUB), jnp.int32),
        pltpu.VMEM((_NSUB, _SUB), jnp.int32),
        pltpu.VMEM((3, _SUB, 128), jnp.float32),
        pltpu.VMEM_SHARED((_AR, 128), jnp.float32),
        pltpu.SemaphoreType.DMA,
        pltpu.SemaphoreType.DMA,
        pltpu.SemaphoreType.DMA,
        pltpu.SemaphoreType.DMA,
        pltpu.SemaphoreType.DMA,
        pltpu.SemaphoreType.DMA,
    ],
)
def _mp_kernel(src2_hbm, dst2_hbm, g_hbm, zeros_hbm, out_hbm,
               sidx_v, didx_v, rows_v, acc_sh,
               gsem0, gsem1, gsem2, ssem0, ssem1, ssem2):
    cid = lax.axis_index("c")
    sid = lax.axis_index("s")
    gsems = [gsem0, gsem1, gsem2]
    ssems = [ssem0, ssem1, ssem2]

    for rnd in range(2):
        base = (cid * 2 + rnd) * _Q
        pltpu.sync_copy(zeros_hbm, acc_sh.at[pl.ds(sid * 792, 792)])
        plsc.subcore_barrier()

        def block(b, _):
            row0 = sid * (_NR // 16) + b * _NSUB
            pltpu.sync_copy(src2_hbm.at[pl.ds(row0, _NSUB)], sidx_v)
            pltpu.sync_copy(dst2_hbm.at[pl.ds(row0, _NSUB)], didx_v)
            for j in range(_NSUB):
                for c in range(_SUB // 16):
                    v = didx_v[j, pl.ds(c * 16, 16)]
                    local = v - base
                    oob = (local < 0) | (local >= _Q)
                    dump = _Q + (v & (_DUMP - 1))
                    didx_v[j, pl.ds(c * 16, 16)] = jnp.where(oob, dump, local)
            g_d = [None] * 3
            s_d = [None] * 3
            for j in range(_NSUB + 1):
                if j < _NSUB:
                    p = j % 3
                    if s_d[p] is not None:
                        s_d[p].wait()
                    g_d[p] = pltpu.async_copy(
                        g_hbm.at[sidx_v.at[j]], rows_v.at[p], gsems[p])
                if j >= 1:
                    q = (j - 1) % 3
                    g_d[q].wait()
                    s_d[q] = pltpu.async_copy(
                        rows_v.at[q], acc_sh.at[didx_v.at[j - 1]],
                        ssems[q], add=True)
            for d in s_d:
                if d is not None:
                    d.wait()
            return 0

        lax.fori_loop(0, _NR // 16 // _NSUB, block, 0)
        plsc.subcore_barrier()
        pltpu.sync_copy(acc_sh.at[pl.ds(sid * 784, 784)],
                        out_hbm.at[pl.ds(base + sid * 784, 784)])
        plsc.subcore_barrier()


_RB = 1000


def _tc1_body(feat_ref, struc_ref, dega_ref, degb_ref, w1_ref, w2_ref,
              emb_ref, embw_ref, g_ref, dis_ref):
    deg = dega_ref[...] + degb_ref[...] + 1.0
    dis = lax.rsqrt(deg)
    gate_row = lax.dot_general(embw_ref[...], emb_ref[...],
                               (((0,), (1,)), ((), ())))
    mask = jnp.where(jax.nn.sigmoid(gate_row) < 0.5, 0.0, 1.0)
    h1 = jnp.dot(feat_ref[...], w1_ref[...], preferred_element_type=jnp.float32)
    h2 = jnp.dot(struc_ref[...] * mask, w2_ref[...],
                 preferred_element_type=jnp.float32)
    g_ref[...] = dis * jnp.concatenate([h1, h2], axis=1)
    dis_ref[...] = dis


def _tc2_body(acc_ref, g_ref, dis_ref, b1_ref, b2_ref,
              feat_ref, struc_ref, g2_ref):
    dis = dis_ref[...]
    s = dis * (acc_ref[...] + g_ref[...])
    feat = s[:, :64] + b1_ref[...]
    struc = s[:, 64:] + b2_ref[...]
    feat_ref[...] = feat
    struc_ref[...] = struc
    g2_ref[...] = dis * jnp.concatenate([feat, struc], axis=1)


def _tc3_body(acc_ref, g2_ref, dis_ref, w3_ref, b3_ref, y_ref):
    s = dis_ref[...] * (acc_ref[...] + g2_ref[...])
    y_ref[...] = jnp.dot(s, w3_ref[...],
                         preferred_element_type=jnp.float32) + b3_ref[...]


def _row_spec(cols):
    return pl.BlockSpec((_RB, cols), lambda i: (i, 0))


def _full_spec(r, c):
    return pl.BlockSpec((r, c), lambda i: (0, 0))


def kernel(feature, structure, edge_index, W1, b1, W2, b2, W3, b3, emb, embW,
           alpha_s, alpha_a):
    pad = _E2 - _E
    src2 = jnp.concatenate(
        [edge_index[0], jnp.zeros((pad,), jnp.int32)]).reshape(_NR, _SUB)
    dst2 = jnp.concatenate(
        [edge_index[1], jnp.full((pad,), _DSTPAD, jnp.int32)]).reshape(_NR, _SUB)
    zeros_init = jnp.zeros((792, 128), jnp.float32)
    structure_p = jnp.pad(structure, ((0, 0), (0, 2)))
    W2p = jnp.pad(W2, ((0, 2), (0, 0)))
    embp = jnp.pad(emb, ((0, 2), (0, 0)))
    total = jnp.abs(alpha_s) + jnp.abs(alpha_a)
    W3s = jnp.concatenate([W3[:64] * (2.0 * alpha_a / total),
                           W3[64:] * (2.0 * alpha_s / total)], axis=0)

    deg_r = _deg_kernel(dst2)
    dega = deg_r[:_N].reshape(_N, 1)
    degb = deg_r[_DEGPAD:_DEGPAD + _N].reshape(_N, 1)

    g, dis = pl.pallas_call(
        _tc1_body,
        grid=(_N // _RB,),
        in_specs=[
            _row_spec(128), _row_spec(8), _row_spec(1), _row_spec(1),
            _full_spec(128, 64), _full_spec(8, 64), _full_spec(8, 64),
            _full_spec(64, 1),
        ],
        out_specs=[_row_spec(128), _row_spec(1)],
        out_shape=[jax.ShapeDtypeStruct((_N, 128), jnp.float32),
                   jax.ShapeDtypeStruct((_N, 1), jnp.float32)],
    )(feature, structure_p, dega, degb, W1, W2p, embp, embW)

    acc1 = _mp_kernel(src2, dst2, g, zeros_init)

    feat_out, struc_out, g2 = pl.pallas_call(
        _tc2_body,
        grid=(_N // _RB,),
        in_specs=[_row_spec(128), _row_spec(128), _row_spec(1),
                  _full_spec(1, 64), _full_spec(1, 64)],
        out_specs=[_row_spec(64), _row_spec(64), _row_spec(128)],
        out_shape=[
            jax.ShapeDtypeStruct((_N, 64), jnp.float32),
            jax.ShapeDtypeStruct((_N, 64), jnp.float32),
            jax.ShapeDtypeStruct((_N, 128), jnp.float32),
        ],
    )(acc1, g, dis, b1.reshape(1, 64), b2.reshape(1, 64))

    acc2 = _mp_kernel(src2, dst2, g2, zeros_init)

    y = pl.pallas_call(
        _tc3_body,
        grid=(_N // _RB,),
        in_specs=[_row_spec(128), _row_spec(128), _row_spec(1),
                  _full_spec(128, 16), _full_spec(1, 16)],
        out_specs=_row_spec(16),
        out_shape=jax.ShapeDtypeStruct((_N, 16), jnp.float32),
    )(acc2, g2, dis, W3s, b3.reshape(1, 16))

    return (feat_out, struc_out, y)
